# baseline (speedup 1.0000x reference)
import functools

import jax
import jax.numpy as jnp
from jax import lax
from jax.experimental import pallas as pl
from jax.experimental.pallas import tpu as pltpu
from jax.experimental.pallas import tpu_sc as plsc

SEQ = 200
DIM = 64
BATCH = 4096
ROWS = BATCH * SEQ
NC, NS = 2, 16
NW = NC * NS
RPW = ROWS // NW
IDXW = 100
IDX_ROWS = ROWS // IDXW
IDX_RPW = RPW // IDXW
BLK = 200
NBLK = RPW // BLK
NBUF = 4
UNROLL = 8

_mesh = plsc.VectorSubcoreMesh(core_axis_name="c", subcore_axis_name="s")


@functools.partial(
    pl.kernel,
    out_type=jax.ShapeDtypeStruct((ROWS, DIM), jnp.float32),
    mesh=_mesh,
    scratch_types=[
        pltpu.VMEM((IDX_RPW, IDXW), jnp.int32),
        pltpu.VMEM((SEQ, DIM), jnp.float32),
        pltpu.VMEM((NBUF, BLK, DIM), jnp.float32),
    ] + [pltpu.SemaphoreType.DMA] * (2 * NBUF),
    compiler_params=pltpu.CompilerParams(use_tc_tiling_on_sc=False),
)
def _embed(idx_hbm, tok_hbm, pos_hbm, out_hbm, idx_v, pos_v, buf_v, *sems):
    gsem = sems[:NBUF]
    ssem = sems[NBUF:]
    wid = lax.axis_index("s") * NC + lax.axis_index("c")
    row0 = wid * RPW

    pltpu.sync_copy(idx_hbm.at[pl.ds(wid * IDX_RPW, IDX_RPW)], idx_v)
    pltpu.sync_copy(pos_hbm, pos_v)

    def gather_desc(i, b, h):
        return pltpu.make_async_copy(
            tok_hbm.at[idx_v.at[2 * i + h]],
            buf_v.at[b, pl.ds(h * IDXW, IDXW)],
            gsem[b])

    def scatter_desc(i, b):
        return pltpu.make_async_copy(
            buf_v.at[b],
            out_hbm.at[pl.ds(row0 + i * BLK, BLK)],
            ssem[b])

    def start_gather(i, b):
        for h in range(2):
            gather_desc(i, b, h).start()

    for b in range(NBUF - 1):
        start_gather(b, b)

    def group(g, _):
        for b in range(NBUF):
            i = g * NBUF + b
            gather_desc(i, b, 0).wait()
            gather_desc(i, b, 1).wait()

            def rows_body(r, _2, b=b):
                for u in range(UNROLL):
                    rr = r * UNROLL + u
                    for c in range(DIM // 16):
                        sl = pl.ds(c * 16, 16)
                        buf_v[b, rr, sl] = buf_v[b, rr, sl] + pos_v[rr, sl]
                return 0

            lax.fori_loop(0, BLK // UNROLL, rows_body, 0)
            scatter_desc(i, b).start()

            j = i + NBUF - 1
            bj = (b + NBUF - 1) % NBUF
            if b == 0:
                @pl.when(g >= 1)
                def _wait_prev():
                    scatter_desc(j - NBUF, bj).wait()
            else:
                scatter_desc(j - NBUF, bj).wait()

            @pl.when(j < NBLK)
            def _prefetch():
                start_gather(j, bj)
        return 0

    lax.fori_loop(0, NBLK // NBUF, group, 0)
    scatter_desc(NBLK - 1, (NBLK - 1) % NBUF).wait()


def kernel(inputs, token_table, pos_table):
    idx = inputs.reshape(IDX_ROWS, IDXW)
    out = _embed(idx, token_table, pos_table)
    return out.reshape(BATCH, SEQ, DIM)


# speedup vs baseline: 1.6195x; 1.6195x over previous
import functools

import jax
import jax.numpy as jnp
from jax import lax
from jax.experimental import pallas as pl
from jax.experimental.pallas import tpu as pltpu
from jax.experimental.pallas import tpu_sc as plsc

SEQ = 200
DIM = 64
BATCH = 4096
NC, NS = 2, 16
NW = NC * NS
BW = BATCH // NW
NBUF = 4
PITCH = 129
DT = DIM // 8

_mesh = plsc.VectorSubcoreMesh(core_axis_name="c", subcore_axis_name="s")


@functools.partial(
    pl.kernel,
    out_type=jax.ShapeDtypeStruct((SEQ, DT, NW, 8, 128), jnp.float32),
    mesh=_mesh,
    scratch_types=[
        pltpu.VMEM((SEQ, BW), jnp.int32),
        pltpu.VMEM((SEQ, DIM), jnp.float32),
        pltpu.VMEM((NBUF, BW, DIM), jnp.float32),
        pltpu.VMEM((2, DT, 8, PITCH), jnp.float32),
    ] + [pltpu.SemaphoreType.DMA] * (NBUF + 2),
    compiler_params=pltpu.CompilerParams(
        use_tc_tiling_on_sc=False, needs_layout_passes=False),
)
def _embed(idx_hbm, tok_hbm, pos_hbm, out_hbm, idx_v, pos_v, rows_v,
           stage_v, *sems):
    gsem = sems[:NBUF]
    ssem = sems[NBUF:]
    wid = lax.axis_index("s") * NC + lax.axis_index("c")

    pltpu.sync_copy(idx_hbm.at[:, pl.ds(wid * BW, BW)], idx_v)
    pltpu.sync_copy(pos_hbm, pos_v)

    def gather_desc(s, b):
        return pltpu.make_async_copy(
            tok_hbm.at[idx_v.at[s]], rows_v.at[b], gsem[b])

    def out_desc(s, p):
        return pltpu.make_async_copy(
            stage_v.at[p, :, :, pl.ds(0, 128)],
            out_hbm.at[s, :, wid], ssem[p])

    iota = lax.iota(jnp.int32, 16)
    dt_idx = [(iota + c * 16) // 8 for c in range(DIM // 16)]
    d8_idx = [(iota + c * 16) % 8 for c in range(DIM // 16)]

    for s0 in range(NBUF - 1):
        gather_desc(s0, s0).start()

    def group(g, _):
        for b in range(NBUF):
            s = g * NBUF + b
            p = b % 2
            gather_desc(s, b).wait()

            nxt = s + NBUF - 1
            @pl.when(nxt < SEQ)
            def _prefetch():
                gather_desc(nxt, (b + NBUF - 1) % NBUF).start()

            if b < 2:
                @pl.when(g >= 1)
                def _wait_stage():
                    out_desc(s - 2, p).wait()
            else:
                out_desc(s - 2, p).wait()

            pos4 = [pos_v[s, pl.ds(c * 16, 16)] for c in range(DIM // 16)]

            @plsc.parallel_loop(0, BW, 1, unroll=8)
            def _rows(r, b=b, p=p):
                bsplat = jnp.full((16,), r, jnp.int32)
                for c in range(DIM // 16):
                    v = rows_v[b, r, pl.ds(c * 16, 16)] + pos4[c]
                    plsc.store_scatter(
                        stage_v.at[p], [dt_idx[c], d8_idx[c], bsplat], v)

            out_desc(s, p).start()
        return 0

    lax.fori_loop(0, SEQ // NBUF, group, 0)
    for s in (SEQ - 2, SEQ - 1):
        out_desc(s, s % 2).wait()


def kernel(inputs, token_table, pos_table):
    idx = inputs.T
    out5 = _embed(idx, token_table, pos_table)
    return out5.transpose(2, 4, 0, 1, 3).reshape(BATCH, SEQ, DIM)


# speedup vs baseline: 3.3093x; 2.0434x over previous
import functools

import jax
import jax.numpy as jnp
from jax import lax
from jax.experimental import pallas as pl
from jax.experimental.pallas import tpu as pltpu
from jax.experimental.pallas import tpu_sc as plsc

SEQ = 200
DIM = 64
BATCH = 4096
NC, NS = 2, 16
NW = NC * NS
BW = BATCH // NW
NBUF = 4
PITCH = 129
DT = DIM // 8
TBLK = 32768
LB = TBLK.bit_length() - 1
NTB = -(-1000000 // TBLK)
TROWS = NTB * (TBLK // 2) * 2

_mesh = plsc.VectorSubcoreMesh(core_axis_name="c", subcore_axis_name="s")


def _detile_body(x_ref, o_ref):
    y = x_ref[...].T
    o_ref[:, 0:DIM] = y[0:TBLK // 2]
    o_ref[:, DIM:2 * DIM] = y[TBLK // 2:TBLK]


def _detile(tab_t):
    return pl.pallas_call(
        _detile_body,
        grid=(NTB,),
        in_specs=[pl.BlockSpec((DIM, TBLK), lambda i: (0, i))],
        out_specs=pl.BlockSpec((TBLK // 2, 2 * DIM), lambda i: (i, 0)),
        out_shape=jax.ShapeDtypeStruct((NTB * (TBLK // 2), 2 * DIM),
                                       jnp.float32),
    )(tab_t)


@functools.partial(
    pl.kernel,
    out_type=jax.ShapeDtypeStruct((SEQ, DT, NW, 8, 128), jnp.float32),
    mesh=_mesh,
    scratch_types=[
        pltpu.VMEM((SEQ, BW), jnp.int32),
        pltpu.VMEM((SEQ, DIM), jnp.float32),
        pltpu.VMEM((NBUF, BW, DIM), jnp.float32),
        pltpu.VMEM((NBUF, DT, 8, PITCH), jnp.float32),
    ] + [pltpu.SemaphoreType.DMA] * (2 * NBUF),
    compiler_params=pltpu.CompilerParams(
        use_tc_tiling_on_sc=False, needs_layout_passes=False),
)
def _embed(idx_hbm, tok_hbm, pos_hbm, out_hbm, idx_v, pos_v, rows_v,
           stage_v, *sems):
    gsem = sems[:NBUF]
    ssem = sems[NBUF:]
    wid = lax.axis_index("s") * NC + lax.axis_index("c")

    pltpu.sync_copy(idx_hbm.at[:, pl.ds(wid * BW, BW)], idx_v)
    pltpu.sync_copy(pos_hbm, pos_v)

    def _xform(k, _):
        for j in range(BW // 16):
            sl = pl.ds(j * 16, 16)
            t = idx_v[k, sl]
            idx_v[k, sl] = ((t >> LB) << LB) | ((t & (TBLK // 2 - 1)) << 1) | (
                (t >> (LB - 1)) & 1)
        return 0

    lax.fori_loop(0, SEQ, _xform, 0)

    def gather_desc(s, b):
        return pltpu.make_async_copy(
            tok_hbm.at[idx_v.at[s]], rows_v.at[b], gsem[b])

    def out_desc(s, p):
        return pltpu.make_async_copy(
            stage_v.at[p, :, :, pl.ds(0, 128)],
            out_hbm.at[s, :, wid], ssem[p])

    iota = lax.iota(jnp.int32, 16)
    dt_idx = [(iota + c * 16) // 8 for c in range(DIM // 16)]
    d8_idx = [(iota + c * 16) % 8 for c in range(DIM // 16)]

    for s0 in range(NBUF - 1):
        gather_desc(s0, s0).start()

    def group(g, _):
        for b in range(NBUF):
            s = g * NBUF + b
            p = b
            gather_desc(s, b).wait()

            nxt = s + NBUF - 1
            @pl.when(nxt < SEQ)
            def _prefetch():
                gather_desc(nxt, (b + NBUF - 1) % NBUF).start()

            @pl.when(g >= 1)
            def _wait_stage():
                out_desc(s - NBUF, p).wait()

            pos4 = [pos_v[s, pl.ds(c * 16, 16)] for c in range(DIM // 16)]

            @plsc.parallel_loop(0, BW, 1, unroll=16)
            def _rows(r, b=b, p=p):
                bsplat = jnp.full((16,), r, jnp.int32)
                for c in range(DIM // 16):
                    v = rows_v[b, r, pl.ds(c * 16, 16)] + pos4[c]
                    plsc.store_scatter(
                        stage_v.at[p], [dt_idx[c], d8_idx[c], bsplat], v)

            out_desc(s, p).start()
        return 0

    lax.fori_loop(0, SEQ // NBUF, group, 0)
    for s in range(SEQ - NBUF, SEQ):
        out_desc(s, s % NBUF).wait()


def kernel(inputs, token_table, pos_table):
    idx = inputs.T
    tab = _detile(token_table.T).reshape(TROWS, DIM)
    out5 = _embed(idx, tab, pos_table)
    return out5.transpose(2, 4, 0, 1, 3).reshape(BATCH, SEQ, DIM)
